# baseline (speedup 1.0000x reference)
import functools

import jax
import jax.numpy as jnp
from jax import lax
from jax.experimental import pallas as pl
from jax.experimental.pallas import tpu as pltpu
from jax.experimental.pallas import tpu_sc as plsc

VOCAB = 1000000
EMBED = 64
B, L = 4096, 200
N_ROWS = B * L

NC, NS = 2, 16
NW = NC * NS
ROWS_PER_W = N_ROWS // NW
CHUNK = 128
N_CHUNKS = ROWS_PER_W // CHUNK


@functools.partial(
    pl.kernel,
    out_type=jax.ShapeDtypeStruct((N_ROWS, EMBED), jnp.float32),
    mesh=plsc.VectorSubcoreMesh(
        core_axis_name="c", subcore_axis_name="s", num_cores=NC,
        num_subcores=NS),
    scratch_types=[
        pltpu.VMEM((CHUNK,), jnp.int32),
        pltpu.VMEM((CHUNK, EMBED), jnp.float32),
        pltpu.SemaphoreType.DMA,
    ],
    compiler_params=pltpu.CompilerParams(use_tc_tiling_on_sc=False),
)
def _embed_gather(table_hbm, ids_hbm, out_hbm, idx_v, rows_v, sem):
    wid = lax.axis_index("s") * NC + lax.axis_index("c")
    base = wid * ROWS_PER_W

    @pl.loop(0, N_CHUNKS)
    def _chunk(i):
        off = base + i * CHUNK
        pltpu.sync_copy(ids_hbm.at[pl.ds(off, CHUNK)], idx_v)
        pltpu.async_copy(table_hbm.at[idx_v], rows_v, sem).wait()
        pltpu.sync_copy(rows_v, out_hbm.at[pl.ds(off, CHUNK)])


@jax.jit
def kernel(input_ids, table):
    ids = input_ids.reshape(-1).astype(jnp.int32)
    out = _embed_gather(table, ids)
    return out.reshape(B, L, EMBED)


# speedup vs baseline: 1.1754x; 1.1754x over previous
import functools

import jax
import jax.numpy as jnp
from jax import lax
from jax.experimental import pallas as pl
from jax.experimental.pallas import tpu as pltpu
from jax.experimental.pallas import tpu_sc as plsc

VOCAB = 1000000
EMBED = 64
B, L = 4096, 200
N_ROWS = B * L

NC, NS = 2, 16
NW = NC * NS
ROWS_PER_W = N_ROWS // NW
CHUNK = 1024
N_CHUNKS = ROWS_PER_W // CHUNK


@functools.partial(
    pl.kernel,
    out_type=jax.ShapeDtypeStruct((N_ROWS, EMBED), jnp.float32),
    mesh=plsc.VectorSubcoreMesh(
        core_axis_name="c", subcore_axis_name="s", num_cores=NC,
        num_subcores=NS),
    scratch_types=[
        pltpu.VMEM((CHUNK,), jnp.int32),
        pltpu.VMEM((CHUNK, EMBED), jnp.float32),
        pltpu.SemaphoreType.DMA,
    ],
    compiler_params=pltpu.CompilerParams(use_tc_tiling_on_sc=False),
)
def _embed_gather(table_hbm, ids_hbm, out_hbm, idx_v, rows_v, sem):
    wid = lax.axis_index("s") * NC + lax.axis_index("c")
    base = wid * ROWS_PER_W

    @pl.loop(0, N_CHUNKS)
    def _chunk(i):
        off = base + i * CHUNK
        pltpu.sync_copy(ids_hbm.at[pl.ds(off, CHUNK)], idx_v)
        pltpu.async_copy(table_hbm.at[idx_v], rows_v, sem).wait()
        pltpu.sync_copy(rows_v, out_hbm.at[pl.ds(off, CHUNK)])


@jax.jit
def kernel(input_ids, table):
    ids = input_ids.reshape(-1).astype(jnp.int32)
    out = _embed_gather(table, ids)
    return out.reshape(B, L, EMBED)


# speedup vs baseline: 1.1918x; 1.0139x over previous
import functools

import jax
import jax.numpy as jnp
from jax import lax
from jax.experimental import pallas as pl
from jax.experimental.pallas import tpu as pltpu
from jax.experimental.pallas import tpu_sc as plsc

VOCAB = 1000000
EMBED = 64
B, L = 4096, 200
N_ROWS = B * L

NC, NS = 2, 16
NW = NC * NS
ROWS_PER_W = N_ROWS // NW
G = 400
NBUF = 4
NSLAB = ROWS_PER_W // G


@functools.partial(
    pl.kernel,
    out_type=jax.ShapeDtypeStruct((N_ROWS, EMBED), jnp.float32),
    mesh=plsc.VectorSubcoreMesh(
        core_axis_name="c", subcore_axis_name="s", num_cores=NC,
        num_subcores=NS),
    scratch_types=[
        pltpu.VMEM((ROWS_PER_W,), jnp.int32),
        pltpu.VMEM((NBUF, G, EMBED), jnp.float32),
    ] + [pltpu.SemaphoreType.DMA] * (2 * NBUF),
    compiler_params=pltpu.CompilerParams(use_tc_tiling_on_sc=False),
)
def _embed_gather(table_hbm, ids_hbm, out_hbm, idx_v, rows_v, *sems):
    sem_g = sems[:NBUF]
    sem_s = sems[NBUF:]
    wid = lax.axis_index("s") * NC + lax.axis_index("c")
    base = wid * ROWS_PER_W

    pltpu.sync_copy(ids_hbm.at[pl.ds(base, ROWS_PER_W)], idx_v)

    def g_desc(s, b):
        return pltpu.make_async_copy(
            table_hbm.at[idx_v.at[pl.ds(s * G, G)]], rows_v.at[b], sem_g[b])

    def s_desc(s, b):
        return pltpu.make_async_copy(
            rows_v.at[b], out_hbm.at[pl.ds(base + s * G, G)], sem_s[b])

    for j in range(NBUF - 1):
        g_desc(j, j).start()
    g_desc(NBUF - 1, NBUF - 1).start()
    g_desc(0, 0).wait()
    s_desc(0, 0).start()

    @pl.loop(0, (NSLAB - NBUF) // NBUF)
    def _steady(i):
        for j in range(NBUF):
            s = NBUF * i + 1 + j
            bs = (1 + j) % NBUF
            bp = j
            s_desc(s - 1, bp).wait()
            g_desc(s + NBUF - 1, bp).start()
            g_desc(s, bs).wait()
            s_desc(s, bs).start()

    for j in range(NBUF - 1):
        s = NSLAB - NBUF + 1 + j
        s_desc(s - 1, (s - 1) % NBUF).wait()
        g_desc(s, s % NBUF).wait()
        s_desc(s, s % NBUF).start()
    s_desc(NSLAB - 1, (NSLAB - 1) % NBUF).wait()


@jax.jit
def kernel(input_ids, table):
    ids = input_ids.reshape(-1).astype(jnp.int32)
    out = _embed_gather(table, ids)
    return out.reshape(B, L, EMBED)


# speedup vs baseline: 1.5832x; 1.3284x over previous
import functools

import jax
import jax.numpy as jnp
from jax import lax
from jax.experimental import pallas as pl
from jax.experimental.pallas import tpu as pltpu
from jax.experimental.pallas import tpu_sc as plsc

VOCAB = 1000000
EMBED = 64
B, L = 4096, 200
N_ROWS = B * L

NC, NS = 2, 16
NW = NC * NS
ROWS_PER_W = N_ROWS // NW
G = 400
NBUF = 4
NSLAB = ROWS_PER_W // G

_MESH = plsc.VectorSubcoreMesh(
    core_axis_name="c", subcore_axis_name="s", num_cores=NC, num_subcores=NS)


@functools.partial(
    pl.kernel,
    out_type=jax.ShapeDtypeStruct((N_ROWS, 2 * EMBED), jnp.float32),
    mesh=_MESH,
    scratch_types=[
        pltpu.VMEM((ROWS_PER_W,), jnp.int32),
        pltpu.VMEM((NBUF, G, EMBED), jnp.float32),
    ] + [pltpu.SemaphoreType.DMA] * (2 * NBUF),
    compiler_params=pltpu.CompilerParams(use_tc_tiling_on_sc=False),
)
def _embed_gather(table_hbm, ids_hbm, out_hbm, idx_v, rows_v, *sems):
    sem_g = sems[:NBUF]
    sem_s = sems[NBUF:]
    wid = lax.axis_index("s") * NC + lax.axis_index("c")
    base = wid * ROWS_PER_W

    pltpu.sync_copy(ids_hbm.at[pl.ds(base, ROWS_PER_W)], idx_v)

    def g_desc(s, b):
        return pltpu.make_async_copy(
            table_hbm.at[idx_v.at[pl.ds(s * G, G)]], rows_v.at[b], sem_g[b])

    def s_desc(s, b):
        return pltpu.make_async_copy(
            rows_v.at[b],
            out_hbm.at[pl.ds(base + s * G, G), pl.ds(0, EMBED)],
            sem_s[b])

    for j in range(NBUF - 1):
        g_desc(j, j).start()
    g_desc(NBUF - 1, NBUF - 1).start()
    g_desc(0, 0).wait()
    s_desc(0, 0).start()

    @pl.loop(0, (NSLAB - NBUF) // NBUF)
    def _steady(i):
        for j in range(NBUF):
            s = NBUF * i + 1 + j
            bs = (1 + j) % NBUF
            bp = j
            s_desc(s - 1, bp).wait()
            g_desc(s + NBUF - 1, bp).start()
            g_desc(s, bs).wait()
            s_desc(s, bs).start()

    for j in range(NBUF - 1):
        s = NSLAB - NBUF + 1 + j
        s_desc(s - 1, (s - 1) % NBUF).wait()
        g_desc(s, s % NBUF).wait()
        s_desc(s, s % NBUF).start()
    s_desc(NSLAB - 1, (NSLAB - 1) % NBUF).wait()


@jax.jit
def kernel(input_ids, table):
    ids = input_ids.reshape(-1).astype(jnp.int32)
    out = _embed_gather(table, ids)
    return out[:, :EMBED].reshape(B, L, EMBED)
